# baseline (speedup 1.0000x reference)
import jax
import jax.numpy as jnp
from jax.experimental import pallas as pl

_N_DIMS = 26


def _onehot_body(idx_ref, out_ref):
    idx = idx_ref[...]
    r, l = idx.shape
    iota = jax.lax.broadcasted_iota(jnp.int32, (r, l, _N_DIMS), 2)
    out_ref[...] = (idx[..., None] == iota).astype(jnp.int32)


def kernel(idxs_vec):
    b, l = idxs_vec.shape
    r = 128
    return pl.pallas_call(
        _onehot_body,
        grid=(b // r,),
        in_specs=[pl.BlockSpec((r, l), lambda i: (i, 0))],
        out_specs=pl.BlockSpec((r, l, _N_DIMS), lambda i: (i, 0, 0)),
        out_shape=jax.ShapeDtypeStruct((b, l, _N_DIMS), jnp.int32),
    )(idxs_vec)


# speedup vs baseline: 1.4900x; 1.4900x over previous
import jax
import jax.numpy as jnp
from jax.experimental import pallas as pl

_N = 26


def _onehot_body(idx_ref, sel_ref, kpat_ref, out_ref):
    x = idx_ref[...].astype(jnp.bfloat16)
    y = jax.lax.dot_general(x, sel_ref[...], (((1,), (0,)), ((), ())),
                            preferred_element_type=jnp.float32)
    out_ref[...] = jnp.where(y == kpat_ref[...], 1, 0).astype(jnp.int32)


def kernel(idxs_vec):
    b, l = idxs_vec.shape
    f = l * _N
    r = 256
    sel = jnp.repeat(jnp.eye(l, dtype=jnp.bfloat16), _N, axis=1)
    kpat = (jnp.arange(f, dtype=jnp.int32) % _N).astype(jnp.float32)[None, :]
    out2d = pl.pallas_call(
        _onehot_body,
        grid=(b // r,),
        in_specs=[
            pl.BlockSpec((r, l), lambda i: (i, 0)),
            pl.BlockSpec((l, f), lambda i: (0, 0)),
            pl.BlockSpec((1, f), lambda i: (0, 0)),
        ],
        out_specs=pl.BlockSpec((r, f), lambda i: (i, 0)),
        out_shape=jax.ShapeDtypeStruct((b, f), jnp.int32),
    )(idxs_vec, sel, kpat)
    return out2d.reshape(b, l, _N)


# speedup vs baseline: 13.9625x; 9.3708x over previous
import jax
import jax.numpy as jnp
from jax.experimental import pallas as pl

_N = 26


def _onehot_body(idxt_ref, out_ref):
    x = idxt_ref[...]
    l, c = x.shape
    k = jax.lax.broadcasted_iota(jnp.int32, (_N, l, c), 0)
    out_ref[...] = jnp.where(x[None, :, :] == k, 1, 0).astype(jnp.int32)


def kernel(idxs_vec):
    b, l = idxs_vec.shape
    idxt = idxs_vec.T
    c = 512
    out3 = pl.pallas_call(
        _onehot_body,
        grid=(b // c,),
        in_specs=[pl.BlockSpec((l, c), lambda i: (0, i))],
        out_specs=pl.BlockSpec((_N, l, c), lambda i: (0, 0, i)),
        out_shape=jax.ShapeDtypeStruct((_N, l, b), jnp.int32),
    )(idxt)
    return jnp.transpose(out3, (2, 1, 0))


# speedup vs baseline: 14.6109x; 1.0464x over previous
import jax
import jax.numpy as jnp
from jax.experimental import pallas as pl

_N = 26


def _onehot_body(idxt_ref, out_ref):
    x = idxt_ref[...]
    k = pl.program_id(0)
    out_ref[...] = jnp.where(x[None, :, :] == k, 1, 0).astype(jnp.int32)


def kernel(idxs_vec):
    b, l = idxs_vec.shape
    idxt = idxs_vec.T
    out3 = pl.pallas_call(
        _onehot_body,
        grid=(_N,),
        in_specs=[pl.BlockSpec((l, b), lambda k: (0, 0))],
        out_specs=pl.BlockSpec((1, l, b), lambda k: (k, 0, 0)),
        out_shape=jax.ShapeDtypeStruct((_N, l, b), jnp.int32),
    )(idxt)
    return jnp.transpose(out3, (2, 1, 0))
